# baseline (speedup 1.0000x reference)
import functools

import jax
import jax.numpy as jnp
from jax import lax
from jax.experimental import pallas as pl
from jax.experimental.pallas import tpu as pltpu
from jax.experimental.pallas import tpu_sc as plsc

N = 10000
E = 320000
DIN = 128
H = 8
D = 16
HD = H * D
KVC = 2 * HD
ACC = HD + 16
NSUB = 16
NW = 2 * NSUB
ET = E // NW
CH = 40
NCH = ET // CH
NSEG = 5
SCH = NCH // NSEG
RPS = N // NSUB


def _proj_body(nf_ref, wq_ref, bq_ref, wk_ref, bk_ref, wv_ref, bv_ref,
               q_ref, kv_ref):
    nf = nf_ref[...]
    q_ref[...] = jnp.dot(nf, wq_ref[...],
                         preferred_element_type=jnp.float32) + bq_ref[...]
    k = (jnp.dot(nf, wk_ref[...],
                 preferred_element_type=jnp.float32) + bk_ref[...]) * 0.25
    v = jnp.dot(nf, wv_ref[...], preferred_element_type=jnp.float32) + bv_ref[...]
    kv_ref[:, :HD] = k
    kv_ref[:, HD:] = v


def _epi_body(p_ref, e_ref, o_ref):
    p = p_ref[0] + p_ref[1]
    zrep = jnp.dot(p, e_ref[...], preferred_element_type=jnp.float32)
    o_ref[...] = p[:, :HD] / zrep


def _interleave_pairs(x):
    s = x.shape[:-1]
    return x.reshape(*s, H // 2, 2, D).swapaxes(-1, -2).reshape(*s, HD)


_mesh = plsc.VectorSubcoreMesh(core_axis_name="c", subcore_axis_name="s")


@functools.partial(
    pl.kernel,
    mesh=_mesh,
    compiler_params=pltpu.CompilerParams(needs_layout_passes=False,
                                         use_tc_tiling_on_sc=False),
    out_type=jax.ShapeDtypeStruct((2, N, ACC), jnp.float32),
    scratch_types=[
        pltpu.VMEM((SCH, CH), jnp.int32),
        pltpu.VMEM((SCH, CH), jnp.int32),
        pltpu.VMEM((CH, KVC), jnp.bfloat16),
        pltpu.VMEM((CH, KVC), jnp.bfloat16),
        pltpu.VMEM((CH, HD), jnp.bfloat16),
        pltpu.VMEM((CH, HD), jnp.bfloat16),
        pltpu.VMEM((CH, ACC), jnp.float32),
        pltpu.VMEM((CH, ACC), jnp.float32),
        pltpu.SemaphoreType.DMA,
        pltpu.SemaphoreType.DMA,
        pltpu.SemaphoreType.DMA,
        pltpu.SemaphoreType.DMA,
        pltpu.SemaphoreType.DMA,
        pltpu.SemaphoreType.DMA,
        pltpu.VMEM_SHARED((N, ACC), jnp.float32),
    ],
)
def _sc_edge_kernel(kv_hbm, q_hbm, src_hbm, dst_hbm, out_hbm,
                    src_v, dst_v, kvg0, kvg1, qg0, qg1, outc0, outc1,
                    skv0, skv1, sq0, sq1, ssc0, ssc1, acc_sh):
    cid = lax.axis_index("c")
    sid = lax.axis_index("s")
    wid = cid * NSUB + sid
    kvg = (kvg0, kvg1)
    qg = (qg0, qg1)
    outc = (outc0, outc1)
    skv = (skv0, skv1)
    sq = (sq0, sq1)
    ssc = (ssc0, ssc1)

    zero16 = jnp.zeros((16,), jnp.float32)

    def _zb(i, carry):
        for k in range(ACC // 16):
            outc0[i, pl.ds(16 * k, 16)] = zero16
        return carry

    lax.fori_loop(0, CH, _zb, 0)
    base = sid * RPS

    def _zc(i, carry):
        pltpu.sync_copy(outc0, acc_sh.at[pl.ds(base + CH * i, CH)])
        return carry

    lax.fori_loop(0, RPS // CH, _zc, 0)
    pltpu.sync_copy(outc0.at[pl.ds(0, RPS - CH * (RPS // CH))],
                    acc_sh.at[pl.ds(base + CH * (RPS // CH),
                                    RPS - CH * (RPS // CH))])
    plsc.subcore_barrier()

    iota = lax.iota(jnp.int32, 16)
    masks = [jnp.where(iota == h, jnp.float32(1), jnp.float32(0))
             for h in range(H)]
    mask0 = masks[0]

    def _segment(seg, carry):
        pltpu.sync_copy(src_hbm.at[wid, pl.ds(seg * SCH, SCH)], src_v)
        pltpu.sync_copy(dst_hbm.at[wid, pl.ds(seg * SCH, SCH)], dst_v)
        pltpu.async_copy(kv_hbm.at[src_v.at[0]], kvg0, skv0)
        pltpu.async_copy(q_hbm.at[dst_v.at[0]], qg0, sq0)

        def _pair(j2, c1_):
            for b in range(2):
                j = 2 * j2 + b
                jn = jnp.where(j + 1 < SCH, j + 1, SCH - 1)
                pltpu.async_copy(kv_hbm.at[src_v.at[jn]], kvg[1 - b],
                                 skv[1 - b])
                pltpu.async_copy(q_hbm.at[dst_v.at[jn]], qg[1 - b],
                                 sq[1 - b])
                pltpu.make_async_copy(kv_hbm.at[src_v.at[j]], kvg[b],
                                      skv[b]).wait()
                pltpu.make_async_copy(q_hbm.at[dst_v.at[j]], qg[b],
                                      sq[b]).wait()

                @pl.when(j2 >= 1)
                def _():
                    pltpu.make_async_copy(outc[b], acc_sh.at[dst_v.at[j]],
                                          ssc[b]).wait()

                @plsc.parallel_loop(0, CH, 1, unroll=8)
                def _edge(e):
                    sv = jnp.zeros((16,), jnp.float32)
                    for hp in range(H // 2):
                        k32 = kvg[b][e, pl.ds(32 * hp, 32)]
                        q32 = qg[b][e, pl.ds(32 * hp, 32)]
                        v32 = kvg[b][e, pl.ds(HD + 32 * hp, 32)]
                        kab = plsc.unpack(k32,
                                          format=plsc.PackFormat.INTERLEAVED)
                        qab = plsc.unpack(q32,
                                          format=plsc.PackFormat.INTERLEAVED)
                        vab = plsc.unpack(v32,
                                          format=plsc.PackFormat.INTERLEAVED)
                        for m in range(2):
                            h = 2 * hp + m
                            c1 = plsc.cumsum(kab[m] * qab[m])
                            bc = plsc.cumsum(jnp.flip(c1, axis=0) * mask0)
                            bc = jnp.minimum(jnp.maximum(bc, -5.0), 5.0)
                            ev = jnp.exp(bc)
                            outc[b][e, pl.ds(16 * h, 16)] = vab[m] * ev
                            sv = sv + masks[h] * ev
                    outc[b][e, pl.ds(HD, 16)] = sv

                pltpu.async_copy(outc[b], acc_sh.at[dst_v.at[j]], ssc[b],
                                 add=True)
            return c1_

        lax.fori_loop(0, SCH // 2, _pair, 0)
        pltpu.make_async_copy(kv_hbm.at[src_v.at[0]], kvg0, skv0).wait()
        pltpu.make_async_copy(q_hbm.at[dst_v.at[0]], qg0, sq0).wait()
        pltpu.make_async_copy(outc0, acc_sh.at[dst_v.at[0]], ssc0).wait()
        pltpu.make_async_copy(outc1, acc_sh.at[dst_v.at[0]], ssc1).wait()
        return carry

    lax.fori_loop(0, NSEG, _segment, 0)

    plsc.subcore_barrier()
    pltpu.sync_copy(acc_sh.at[pl.ds(base, RPS)],
                    out_hbm.at[cid, pl.ds(base, RPS)])


def kernel(node_feats, edge_index, Wq, bq, Wk, bk, Wv, bv):
    q1, kv1 = pl.pallas_call(
        _proj_body,
        out_shape=(jax.ShapeDtypeStruct((N, HD), jnp.float32),
                   jax.ShapeDtypeStruct((N, KVC), jnp.float32)),
    )(node_feats, Wq, bq.reshape(1, HD), Wk, bk.reshape(1, HD),
      Wv, bv.reshape(1, HD))

    qb = _interleave_pairs(q1).astype(jnp.bfloat16)
    kvb = jnp.concatenate(
        [_interleave_pairs(kv1[:, :HD]).astype(jnp.bfloat16),
         _interleave_pairs(kv1[:, HD:]).astype(jnp.bfloat16)], axis=-1)

    src = edge_index[0].reshape(NW, NCH, CH)
    dst = edge_index[1].reshape(NW, NCH, CH)
    partial = _sc_edge_kernel(kvb, qb, src, dst)

    expand = jnp.zeros((ACC, HD), jnp.float32).at[HD:HD + H].set(
        jnp.repeat(jnp.eye(H, dtype=jnp.float32), D, axis=1))
    out = pl.pallas_call(
        _epi_body,
        out_shape=jax.ShapeDtypeStruct((N, HD), jnp.float32),
    )(partial, expand)
    return out.reshape(N, H, D)


# speedup vs baseline: 3.4232x; 3.4232x over previous
import functools

import jax
import jax.numpy as jnp
from jax import lax
from jax.experimental import pallas as pl
from jax.experimental.pallas import tpu as pltpu
from jax.experimental.pallas import tpu_sc as plsc

N = 10000
E = 320000
DIN = 128
H = 8
D = 16
HD = H * D
KVC = 2 * HD
ACC = HD + 16
NSUB = 16
NW = 2 * NSUB
ET = E // NW
CH = 40
NCH = ET // CH
NSEG = 5
SCH = NCH // NSEG
RPS = N // NSUB


def _proj_body(nf_ref, wq_ref, bq_ref, wk_ref, bk_ref, wv_ref, bv_ref,
               q_ref, kv_ref):
    nf = nf_ref[...]
    q_ref[...] = jnp.dot(nf, wq_ref[...],
                         preferred_element_type=jnp.float32) + bq_ref[...]
    k = (jnp.dot(nf, wk_ref[...],
                 preferred_element_type=jnp.float32) + bk_ref[...]) * 0.25
    v = jnp.dot(nf, wv_ref[...], preferred_element_type=jnp.float32) + bv_ref[...]
    kv_ref[:, :HD] = k
    kv_ref[:, HD:] = v


def _epi_body(p_ref, e_ref, o_ref):
    p = p_ref[0] + p_ref[1]
    zrep = jnp.dot(p, e_ref[...], preferred_element_type=jnp.float32)
    o_ref[...] = p[:, :HD] / zrep


def _interleave_pairs(x):
    s = x.shape[:-1]
    return x.reshape(*s, H // 2, 2, D).swapaxes(-1, -2).reshape(*s, HD)


_mesh = plsc.VectorSubcoreMesh(core_axis_name="c", subcore_axis_name="s")


@functools.partial(
    pl.kernel,
    mesh=_mesh,
    compiler_params=pltpu.CompilerParams(needs_layout_passes=False,
                                         use_tc_tiling_on_sc=False),
    out_type=jax.ShapeDtypeStruct((2, N, ACC), jnp.float32),
    scratch_types=[
        pltpu.VMEM((SCH, CH), jnp.int32),
        pltpu.VMEM((SCH, CH), jnp.int32),
        pltpu.VMEM((CH, KVC), jnp.bfloat16),
        pltpu.VMEM((CH, KVC), jnp.bfloat16),
        pltpu.VMEM((CH, HD), jnp.bfloat16),
        pltpu.VMEM((CH, HD), jnp.bfloat16),
        pltpu.VMEM((CH, ACC), jnp.float32),
        pltpu.VMEM((CH, ACC), jnp.float32),
        pltpu.SemaphoreType.DMA,
        pltpu.SemaphoreType.DMA,
        pltpu.SemaphoreType.DMA,
        pltpu.SemaphoreType.DMA,
        pltpu.SemaphoreType.DMA,
        pltpu.SemaphoreType.DMA,
        pltpu.VMEM_SHARED((N, ACC), jnp.float32),
    ],
)
def _sc_edge_kernel(kv_hbm, q_hbm, src_hbm, dst_hbm, out_hbm,
                    src_v, dst_v, kvg0, kvg1, qg0, qg1, outc0, outc1,
                    skv0, skv1, sq0, sq1, ssc0, ssc1, acc_sh):
    cid = lax.axis_index("c")
    sid = lax.axis_index("s")
    wid = cid * NSUB + sid
    kvg = (kvg0, kvg1)
    qg = (qg0, qg1)
    outc = (outc0, outc1)
    skv = (skv0, skv1)
    sq = (sq0, sq1)
    ssc = (ssc0, ssc1)

    zero16 = jnp.zeros((16,), jnp.float32)

    def _zb(i, carry):
        for k in range(ACC // 16):
            outc0[i, pl.ds(16 * k, 16)] = zero16
        return carry

    lax.fori_loop(0, CH, _zb, 0)
    base = sid * RPS

    def _zc(i, carry):
        pltpu.sync_copy(outc0, acc_sh.at[pl.ds(base + CH * i, CH)])
        return carry

    lax.fori_loop(0, RPS // CH, _zc, 0)
    pltpu.sync_copy(outc0.at[pl.ds(0, RPS - CH * (RPS // CH))],
                    acc_sh.at[pl.ds(base + CH * (RPS // CH),
                                    RPS - CH * (RPS // CH))])
    plsc.subcore_barrier()

    iota = lax.iota(jnp.int32, 16)
    masks = [jnp.where(iota == h, jnp.float32(1), jnp.float32(0))
             for h in range(H)]
    mask0 = masks[0]

    def _segment(seg, carry):
        pltpu.sync_copy(src_hbm.at[wid, pl.ds(seg * SCH, SCH)], src_v)
        pltpu.sync_copy(dst_hbm.at[wid, pl.ds(seg * SCH, SCH)], dst_v)
        pltpu.async_copy(kv_hbm.at[src_v.at[0]], kvg0, skv0)
        pltpu.async_copy(q_hbm.at[dst_v.at[0]], qg0, sq0)

        def _pair(j2, c1_):
            for b in range(2):
                j = 2 * j2 + b
                jn = jnp.where(j + 1 < SCH, j + 1, SCH - 1)
                pltpu.async_copy(kv_hbm.at[src_v.at[jn]], kvg[1 - b],
                                 skv[1 - b])
                pltpu.async_copy(q_hbm.at[dst_v.at[jn]], qg[1 - b],
                                 sq[1 - b])
                pltpu.make_async_copy(kv_hbm.at[src_v.at[j]], kvg[b],
                                      skv[b]).wait()
                pltpu.make_async_copy(q_hbm.at[dst_v.at[j]], qg[b],
                                      sq[b]).wait()

                @pl.when(j2 >= 1)
                def _():
                    pltpu.make_async_copy(outc[b], acc_sh.at[dst_v.at[j]],
                                          ssc[b]).wait()

                @plsc.parallel_loop(0, CH, 1, unroll=4)
                def _edge(e):
                    sv = jnp.zeros((16,), jnp.float32)
                    for hp in range(H // 2):
                        k32 = kvg[b][e, pl.ds(32 * hp, 32)]
                        q32 = qg[b][e, pl.ds(32 * hp, 32)]
                        v32 = kvg[b][e, pl.ds(HD + 32 * hp, 32)]
                        kab = plsc.unpack(k32,
                                          format=plsc.PackFormat.INTERLEAVED)
                        qab = plsc.unpack(q32,
                                          format=plsc.PackFormat.INTERLEAVED)
                        vab = plsc.unpack(v32,
                                          format=plsc.PackFormat.INTERLEAVED)
                        for m in range(2):
                            h = 2 * hp + m
                            c1 = plsc.cumsum(kab[m] * qab[m])
                            bc = plsc.cumsum(jnp.flip(c1, axis=0) * mask0)
                            bc = jnp.minimum(jnp.maximum(bc, -5.0), 5.0)
                            ev = jnp.exp(bc)
                            outc[b][e, pl.ds(16 * h, 16)] = vab[m] * ev
                            sv = sv + masks[h] * ev
                    outc[b][e, pl.ds(HD, 16)] = sv

                pltpu.async_copy(outc[b], acc_sh.at[dst_v.at[j]], ssc[b],
                                 add=True)
            return c1_

        lax.fori_loop(0, SCH // 2, _pair, 0)
        pltpu.make_async_copy(kv_hbm.at[src_v.at[0]], kvg0, skv0).wait()
        pltpu.make_async_copy(q_hbm.at[dst_v.at[0]], qg0, sq0).wait()
        pltpu.make_async_copy(outc0, acc_sh.at[dst_v.at[0]], ssc0).wait()
        pltpu.make_async_copy(outc1, acc_sh.at[dst_v.at[0]], ssc1).wait()
        return carry

    lax.fori_loop(0, NSEG, _segment, 0)

    plsc.subcore_barrier()
    pltpu.sync_copy(acc_sh.at[pl.ds(base, RPS)],
                    out_hbm.at[cid, pl.ds(base, RPS)])


def kernel(node_feats, edge_index, Wq, bq, Wk, bk, Wv, bv):
    q1, kv1 = pl.pallas_call(
        _proj_body,
        out_shape=(jax.ShapeDtypeStruct((N, HD), jnp.float32),
                   jax.ShapeDtypeStruct((N, KVC), jnp.float32)),
    )(node_feats, Wq, bq.reshape(1, HD), Wk, bk.reshape(1, HD),
      Wv, bv.reshape(1, HD))

    qb = _interleave_pairs(q1).astype(jnp.bfloat16)
    kvb = jnp.concatenate(
        [_interleave_pairs(kv1[:, :HD]).astype(jnp.bfloat16),
         _interleave_pairs(kv1[:, HD:]).astype(jnp.bfloat16)], axis=-1)

    src = edge_index[0].reshape(NW, NCH, CH)
    dst = edge_index[1].reshape(NW, NCH, CH)
    partial = _sc_edge_kernel(kvb, qb, src, dst)

    expand = jnp.zeros((ACC, HD), jnp.float32).at[HD:HD + H].set(
        jnp.repeat(jnp.eye(H, dtype=jnp.float32), D, axis=1))
    out = pl.pallas_call(
        _epi_body,
        out_shape=jax.ShapeDtypeStruct((N, HD), jnp.float32),
    )(partial, expand)
    return out.reshape(N, H, D)
